# baseline (speedup 1.0000x reference)
import jax
import jax.numpy as jnp
from jax.experimental import pallas as pl

ORDER = 16
SUBSAMPLING_FACTOR = 2
NUM_FEATURES = 64
SUB_ORDER = ORDER // SUBSAMPLING_FACTOR


def _copy_kernel(in_ref, out_ref):
    out_ref[...] = in_ref[...]


def kernel(x):
    B, C, H, W = x.shape
    row = NUM_FEATURES * H * W
    xr = x.reshape(B * ORDER, 512, row // 512)
    out = pl.pallas_call(
        _copy_kernel,
        grid=(B, SUB_ORDER),
        in_specs=[
            pl.BlockSpec(
                (1, 512, row // 512),
                lambda b, g: (b * ORDER + g * SUBSAMPLING_FACTOR, 0, 0),
            )
        ],
        out_specs=pl.BlockSpec(
            (1, 512, row // 512), lambda b, g: (b * SUB_ORDER + g, 0, 0)
        ),
        out_shape=jax.ShapeDtypeStruct((B * SUB_ORDER, 512, row // 512), x.dtype),
    )(xr)
    return out.reshape(B, SUB_ORDER * NUM_FEATURES, H, W)
